# baseline (speedup 1.0000x reference)
import jax
import jax.numpy as jnp
from jax.experimental import pallas as pl
from jax.experimental.pallas import tpu as pltpu

_BR = 256
_N_CLASSES = 100
_EPS = 1e-8


def _prep_kernel(amt_ref, out_ref):
    a = amt_ref[...]
    n = jnp.maximum(jnp.sqrt(jnp.sum(a * a, axis=0, keepdims=True)), _EPS)
    out_ref[...] = (a / n).astype(jnp.bfloat16)


def _retrieval_kernel(hvs_ref, amn_ref, out_ref):
    x = hvs_ref[...]
    x_n = x / jnp.maximum(
        jnp.sqrt(jnp.sum(x * x, axis=1, keepdims=True)), _EPS)
    scores = jax.lax.dot_general(
        x_n.astype(jnp.bfloat16), amn_ref[...],
        dimension_numbers=(((1,), (0,)), ((), ())),
        preferred_element_type=jnp.float32,
    )

    m = jnp.max(scores, axis=1, keepdims=True)
    idx = jax.lax.broadcasted_iota(jnp.int32, scores.shape, 1)
    out_ref[...] = jnp.min(jnp.where(scores == m, idx, _N_CLASSES), axis=1,
                           keepdims=True)


@jax.jit
def kernel(hvs, am):
    n_rows, d = hvs.shape
    amt = am.astype(jnp.float32).T
    am_n = pl.pallas_call(
        _prep_kernel,
        out_shape=jax.ShapeDtypeStruct(amt.shape, jnp.bfloat16),
    )(amt)
    out = pl.pallas_call(
        _retrieval_kernel,
        grid=(n_rows // _BR,),
        in_specs=[
            pl.BlockSpec((_BR, d), lambda i: (i, 0)),
            pl.BlockSpec(amt.shape, lambda i: (0, 0)),
        ],
        out_specs=pl.BlockSpec((_BR, 1), lambda i: (i, 0)),
        out_shape=jax.ShapeDtypeStruct((n_rows, 1), jnp.int32),
        compiler_params=pltpu.CompilerParams(
            dimension_semantics=("parallel",)),
    )(hvs, am_n)
    return out.reshape(n_rows)
